# baseline (speedup 1.0000x reference)
import functools

import jax
import jax.numpy as jnp
from jax import lax
from jax.experimental import pallas as pl
from jax.experimental.pallas import tpu as pltpu
from jax.experimental.pallas import tpu_sc as plsc

NUSR = 25000
N = 50000
NP = 50176
TRASH = 50000
E = 800000
EP = 802816
WPT = EP // 16
GRP = 128
GPT = WPT // GRP
NPAIR = GPT // 4
CHD = 512
NCHD = WPT // CHD
CCH = 112
D = 64
DH = 32
B = 4096
NSUB = 16
RPT = NP // NSUB

_mesh = plsc.VectorSubcoreMesh(core_axis_name="c", subcore_axis_name="s")
_f32 = jnp.float32
_sc_params = pltpu.CompilerParams(use_tc_tiling_on_sc=False)
_sc_params_nl = pltpu.CompilerParams(
    use_tc_tiling_on_sc=False, needs_layout_passes=False
)


@functools.partial(
    pl.kernel,
    out_type=[
        jax.ShapeDtypeStruct((2 * NP, DH), _f32),
        jax.ShapeDtypeStruct((NP, 16), _f32),
        jax.ShapeDtypeStruct((NP, 16), _f32),
    ],
    mesh=_mesh,
    scratch_types=[
        pltpu.VMEM((CHD // 128, 128), jnp.int32),
        pltpu.VMEM((CHD // 128, 128), jnp.int32),
        pltpu.VMEM((128, 16), _f32),
        pltpu.VMEM((CCH, 16), _f32),
        pltpu.VMEM((CCH, 16), _f32),
        pltpu.VMEM((CCH, DH), _f32),
        pltpu.VMEM((CCH, DH), _f32),
        pltpu.VMEM_SHARED((NP, 16), _f32),
        pltpu.SemaphoreType.DMA,
        pltpu.SemaphoreType.DMA,
    ],
    compiler_params=_sc_params,
)
def _deg_kernel(
    edges2, onesrow, zeros16, emb_cat, z0, srep, trep,
    idxa, idxb, valb, hbuf, rbuf, ebuf0, ebuf1, hist, semi, sems,
):
    c = lax.axis_index("c")
    s = lax.axis_index("s")
    r0 = s * RPT
    NG = CHD // 128
    pltpu.sync_copy(zeros16.at[pl.ds(r0, RPT)], hist.at[pl.ds(r0, RPT)])
    pltpu.sync_copy(onesrow, valb)
    plsc.subcore_barrier()

    ebase = s * (WPT // 128)

    def stage(idx_c, idx_n, m):
        @pl.when(m > 0)
        def _():
            for j in range(NG):
                pltpu.make_async_copy(valb, hist.at[idx_n.at[j]], sems).wait()

        @pl.when(m < NCHD - 1)
        def _():
            pltpu.async_copy(
                edges2.at[c, pl.ds(ebase + (m + 1) * NG, NG)], idx_n, semi
            )

        @pl.when(m > 0)
        def _():
            pltpu.make_async_copy(
                edges2.at[c, pl.ds(ebase, NG)], idx_c, semi
            ).wait()

        for j in range(NG):
            pltpu.async_copy(valb, hist.at[idx_c.at[j]], sems, add=True)

    pltpu.sync_copy(edges2.at[c, pl.ds(ebase, NG)], idxa)

    def pair(k, carry):
        stage(idxa, idxb, 2 * k)
        stage(idxb, idxa, 2 * k + 1)
        return carry

    lax.fori_loop(0, NCHD // 2, pair, 0)
    for j in range(NG):
        pltpu.make_async_copy(valb, hist.at[idxb.at[j]], sems).wait()
    plsc.subcore_barrier()

    def frsqrt16(x):
        i = lax.bitcast_convert_type(x, jnp.int32)
        i = 0x5F3759DF - lax.shift_right_logical(i, 1)
        y = lax.bitcast_convert_type(i, _f32)
        for _ in range(3):
            y = y * (1.5 - 0.5 * x * y * y)
        return y

    def fchunk(q, carry):
        rq = r0 + q * CCH
        pltpu.sync_copy(hist.at[pl.ds(rq, CCH)], hbuf)

        @pl.when(c == 0)
        def _():
            pltpu.sync_copy(emb_cat.at[pl.ds(rq, CCH)], ebuf0)
            pltpu.sync_copy(emb_cat.at[pl.ds(NP + rq, CCH)], ebuf1)

        def rowf(r, carry2):
            fv = frsqrt16(jnp.maximum(hbuf[r, pl.ds(0, 16)], 1.0))
            rbuf[r, pl.ds(0, 16)] = fv

            @pl.when(c == 0)
            def _():
                for eb in (ebuf0, ebuf1):
                    eb[r, pl.ds(0, 16)] = eb[r, pl.ds(0, 16)] * fv
                    eb[r, pl.ds(16, 16)] = eb[r, pl.ds(16, 16)] * fv

            return carry2

        lax.fori_loop(0, CCH, rowf, 0)

        @pl.when(c == 0)
        def _():
            pltpu.sync_copy(rbuf, srep.at[pl.ds(rq, CCH)])
            pltpu.sync_copy(ebuf0, z0.at[pl.ds(rq, CCH)])
            pltpu.sync_copy(ebuf1, z0.at[pl.ds(NP + rq, CCH)])

        @pl.when(c == 1)
        def _():
            pltpu.sync_copy(rbuf, trep.at[pl.ds(rq, CCH)])

        return carry

    lax.fori_loop(0, RPT // CCH, fchunk, 0)


@functools.partial(
    pl.kernel,
    out_type=[
        jax.ShapeDtypeStruct((2 * NP, DH), _f32),
        jax.ShapeDtypeStruct((2 * NP, DH), _f32),
    ],
    mesh=_mesh,
    scratch_types=[
        pltpu.VMEM((2, 2, 128), jnp.int32),
        pltpu.VMEM((2, 2, 128), jnp.int32),
        pltpu.VMEM((2, 128, DH), _f32),
        pltpu.VMEM((2, 128, DH), _f32),
        pltpu.VMEM((CCH, DH), _f32),
        pltpu.VMEM((CCH, 16), _f32),
        pltpu.VMEM((CCH, 16), _f32),
        pltpu.VMEM_SHARED((NP, DH), _f32),
        pltpu.SemaphoreType.DMA,
        pltpu.SemaphoreType.DMA,
        pltpu.SemaphoreType.DMA,
    ],
    compiler_params=_sc_params,
)
def _spmm_kernel(
    comb, zcat, srep, trep, zerosD, wout, zout,
    cidxa, cidxb, rowsa, rowsb, achunk, schunk, tchunk, acc, semg, sems, semw,
):
    c = lax.axis_index("c")
    s = lax.axis_index("s")
    r0 = s * RPT
    pltpu.sync_copy(zerosD.at[pl.ds(r0, RPT)], acc.at[pl.ds(r0, RPT)])
    plsc.subcore_barrier()

    gbase = s * GPT

    def stage(cidx_c, rows_c, cidx_n, rows_n, m):
        @pl.when(m > 0)
        def _():
            for j in range(2):
                pltpu.make_async_copy(
                    rows_n.at[j], acc.at[cidx_n.at[j, 1]], sems
                ).wait()

        @pl.when(m < 2 * NPAIR - 1)
        def _():
            pltpu.sync_copy(comb.at[c, pl.ds(gbase + 2 * (m + 1), 2)], cidx_n)
            for j in range(2):
                pltpu.async_copy(zcat.at[cidx_n.at[j, 0]], rows_n.at[j], semg)

        for j in range(2):
            pltpu.make_async_copy(
                zcat.at[cidx_c.at[j, 0]], rows_c.at[j], semg
            ).wait()
        for j in range(2):
            pltpu.async_copy(rows_c.at[j], acc.at[cidx_c.at[j, 1]], sems, add=True)

    pltpu.sync_copy(comb.at[c, pl.ds(gbase, 2)], cidxa)
    for j in range(2):
        pltpu.async_copy(zcat.at[cidxa.at[j, 0]], rowsa.at[j], semg)

    def pair(k, carry):
        stage(cidxa, rowsa, cidxb, rowsb, 2 * k)
        stage(cidxb, rowsb, cidxa, rowsa, 2 * k + 1)
        return carry

    lax.fori_loop(0, NPAIR, pair, 0)
    for j in range(2):
        pltpu.make_async_copy(rowsb.at[j], acc.at[cidxb.at[j, 1]], sems).wait()
    plsc.subcore_barrier()

    wo = c * NP + r0
    hw = pltpu.async_copy(
        acc.at[pl.ds(r0, RPT)], wout.at[pl.ds(wo, RPT)], semw
    )

    def cchunk(q, carry):
        rq = r0 + q * CCH
        pltpu.sync_copy(acc.at[pl.ds(rq, CCH)], achunk)
        pltpu.sync_copy(srep.at[pl.ds(rq, CCH)], schunk)
        pltpu.sync_copy(trep.at[pl.ds(rq, CCH)], tchunk)

        def rowf(r, carry2):
            sv = schunk[r, pl.ds(0, 16)] * tchunk[r, pl.ds(0, 16)]
            achunk[r, pl.ds(0, 16)] = achunk[r, pl.ds(0, 16)] * sv
            achunk[r, pl.ds(16, 16)] = achunk[r, pl.ds(16, 16)] * sv
            return carry2

        lax.fori_loop(0, CCH, rowf, 0)
        pltpu.sync_copy(achunk, zout.at[pl.ds(c * NP + rq, CCH)])
        return carry

    lax.fori_loop(0, RPT // CCH, cchunk, 0)
    hw.wait()


@functools.partial(
    pl.kernel,
    out_type=[
        jax.ShapeDtypeStruct((3 * B, D), _f32),
        jax.ShapeDtypeStruct((3 * B, D), _f32),
    ],
    mesh=_mesh,
    scratch_types=[
        pltpu.VMEM((3, 2, 128), jnp.int32),
        pltpu.VMEM((128, D), _f32),
        pltpu.VMEM((128, D), _f32),
        pltpu.VMEM((128, 16), _f32),
        pltpu.VMEM((3, 2, 128, DH), _f32),
        pltpu.SemaphoreType.DMA,
    ],
    compiler_params=_sc_params,
)
def _gather_kernel(
    embT, w1, w2, w3, trep, gidx3, rep_out, embr_out,
    idxb, ebuf, rbuf, tbuf, wbuf, sem,
):
    c = lax.axis_index("c")
    s = lax.axis_index("s")
    w = s * 2 + c
    pltpu.sync_copy(gidx3.at[pl.ds(w * 3, 3)], idxb)
    for j in range(3):
        o = (w * 3 + j) * 128
        hs = [pltpu.async_copy(embT.at[idxb.at[j, 0]], ebuf, sem)]
        hs.append(pltpu.async_copy(trep.at[idxb.at[j, 0]], tbuf, sem))
        for li, wt in enumerate((w1, w2, w3)):
            for h in range(2):
                hs.append(
                    pltpu.async_copy(wt.at[idxb.at[j, h]], wbuf.at[li, h], sem)
                )
        for h in hs:
            h.wait()

        def rowf(r, carry):
            tv = tbuf[r, pl.ds(0, 16)]
            for h in range(2):
                for k in range(2):
                    seg = pl.ds(h * DH + k * 16, 16)
                    sg = pl.ds(k * 16, 16)
                    wsv = (
                        wbuf[0, h, r, sg]
                        + wbuf[1, h, r, sg]
                        + wbuf[2, h, r, sg]
                    )
                    rbuf[r, seg] = (ebuf[r, seg] + tv * wsv) * 0.25
            return carry

        lax.fori_loop(0, 128, rowf, 0)
        pltpu.sync_copy(rbuf, rep_out.at[pl.ds(o, 128)])
        pltpu.sync_copy(ebuf, embr_out.at[pl.ds(o, 128)])


def _l2_body(u_ref, p_ref, n_ref, out_ref):
    out_ref[...] = (
        jnp.sum(u_ref[...] * u_ref[...], axis=1, keepdims=True)
        + jnp.sum(p_ref[...] * p_ref[...], axis=1, keepdims=True)
        + jnp.sum(n_ref[...] * n_ref[...], axis=1, keepdims=True)
    )


_l2_call = pl.pallas_call(
    _l2_body,
    grid=(1,),
    in_specs=[
        pl.BlockSpec((B, D), lambda i: (0, 0)),
        pl.BlockSpec((B, D), lambda i: (1, 0)),
        pl.BlockSpec((B, D), lambda i: (2, 0)),
    ],
    out_specs=pl.BlockSpec((B, 1), lambda i: (0, 0)),
    out_shape=jax.ShapeDtypeStruct((B, 1), _f32),
)


def kernel(users, pos_items, neg_items, edge_index, embedding_weight):
    i32 = jnp.int32
    src = edge_index[0].astype(i32)
    dst = edge_index[1].astype(i32)
    pad = jnp.full((EP - E,), TRASH, dtype=i32)
    srcp = jnp.concatenate([src, pad])
    dstp = jnp.concatenate([dst, pad])
    edges2 = jnp.stack([srcp, dstp]).reshape(2, EP // 128, 128)
    srcg = srcp.reshape(EP // 128, 1, 128)
    dstg = dstp.reshape(EP // 128, 1, 128)
    comb = jnp.stack(
        [
            jnp.concatenate([srcg, dstg], axis=1),
            jnp.concatenate([srcg + NP, dstg], axis=1),
        ]
    )

    emb_p = jnp.concatenate(
        [embedding_weight.astype(_f32), jnp.zeros((NP - N, D), _f32)]
    )
    onesrow = jnp.ones((128, 16), _f32)
    zeros16 = jnp.zeros((NP, 16), _f32)
    zerosD = jnp.zeros((NP, DH), _f32)

    emb_cat = jnp.concatenate([emb_p[:, :DH], emb_p[:, DH:]], axis=0)
    zcat, srep, trep = _deg_kernel(edges2, onesrow, zeros16, emb_cat)
    ws = []
    for _ in range(3):
        w, zcat = _spmm_kernel(comb, zcat, srep, trep, zerosD)
        ws.append(w)

    gidx = jnp.concatenate(
        [
            users.astype(i32),
            NUSR + pos_items.astype(i32),
            NUSR + neg_items.astype(i32),
        ]
    ).reshape(3 * B // 128, 1, 128)
    gidx3 = jnp.concatenate([gidx, gidx + NP], axis=1)
    rep_rows, emb_rows = _gather_kernel(emb_p, ws[0], ws[1], ws[2], trep, gidx3)

    l2 = _l2_call(emb_rows, emb_rows, emb_rows).reshape(B)
    return (
        rep_rows[:B],
        rep_rows[B : 2 * B],
        rep_rows[2 * B :],
        l2,
    )


# speedup vs baseline: 1.0805x; 1.0805x over previous
import functools

import jax
import jax.numpy as jnp
from jax import lax
from jax.experimental import pallas as pl
from jax.experimental.pallas import tpu as pltpu
from jax.experimental.pallas import tpu_sc as plsc

NUSR = 25000
N = 50000
NP = 50176
TRASH = 50000
E = 800000
EP = 802816
WPT = EP // 16
GRP = 128
GPT = WPT // GRP
NPAIR = GPT // 4
CHD = 512
NCHD = WPT // CHD
CCH = 112
D = 64
DH = 32
B = 4096
NSUB = 16
RPT = NP // NSUB

_mesh = plsc.VectorSubcoreMesh(core_axis_name="c", subcore_axis_name="s")
_f32 = jnp.float32
_sc_params = pltpu.CompilerParams(use_tc_tiling_on_sc=False)


@functools.partial(
    pl.kernel,
    out_type=jax.ShapeDtypeStruct((2, NP, 8), _f32),
    mesh=_mesh,
    scratch_types=[
        pltpu.VMEM((CHD // 128, 128), jnp.int32),
        pltpu.VMEM((CHD // 128, 128), jnp.int32),
        pltpu.VMEM((128, 8), _f32),
        pltpu.VMEM_SHARED((NP, 8), _f32),
        pltpu.SemaphoreType.DMA,
        pltpu.SemaphoreType.DMA,
    ],
    compiler_params=_sc_params,
)
def _deg_kernel(edges2, onesrow, zeros8, out, idxa, idxb, valb, hist, semi, sems):
    c = lax.axis_index("c")
    s = lax.axis_index("s")
    r0 = s * RPT
    NG = CHD // 128
    pltpu.sync_copy(zeros8.at[pl.ds(r0, RPT)], hist.at[pl.ds(r0, RPT)])
    pltpu.sync_copy(onesrow, valb)
    plsc.subcore_barrier()

    ebase = s * (WPT // 128)

    def stage(idx_c, idx_n, m):
        @pl.when(m > 0)
        def _():
            for j in range(NG):
                pltpu.make_async_copy(valb, hist.at[idx_n.at[j]], sems).wait()

        @pl.when(m < NCHD - 1)
        def _():
            pltpu.async_copy(
                edges2.at[c, pl.ds(ebase + (m + 1) * NG, NG)], idx_n, semi
            )

        @pl.when(m > 0)
        def _():
            pltpu.make_async_copy(
                edges2.at[c, pl.ds(ebase, NG)], idx_c, semi
            ).wait()

        for j in range(NG):
            pltpu.async_copy(valb, hist.at[idx_c.at[j]], sems, add=True)

    pltpu.sync_copy(edges2.at[c, pl.ds(ebase, NG)], idxa)

    def pair(k, carry):
        stage(idxa, idxb, 2 * k)
        stage(idxb, idxa, 2 * k + 1)
        return carry

    lax.fori_loop(0, NCHD // 2, pair, 0)
    for j in range(NG):
        pltpu.make_async_copy(valb, hist.at[idxb.at[j]], sems).wait()
    plsc.subcore_barrier()
    pltpu.sync_copy(hist.at[pl.ds(r0, RPT)], out.at[c, pl.ds(r0, RPT)])


@functools.partial(
    pl.kernel,
    out_type=[
        jax.ShapeDtypeStruct((2 * NP, DH), _f32),
        jax.ShapeDtypeStruct((2 * NP, DH), _f32),
    ],
    mesh=_mesh,
    scratch_types=[
        pltpu.VMEM((2, 2, 128), jnp.int32),
        pltpu.VMEM((2, 2, 128), jnp.int32),
        pltpu.VMEM((2, 128, DH), _f32),
        pltpu.VMEM((2, 128, DH), _f32),
        pltpu.VMEM((CCH, DH), _f32),
        pltpu.VMEM((CCH, 16), _f32),
        pltpu.VMEM_SHARED((NP, DH), _f32),
        pltpu.SemaphoreType.DMA,
        pltpu.SemaphoreType.DMA,
        pltpu.SemaphoreType.DMA,
    ],
    compiler_params=_sc_params,
)
def _spmm_kernel(
    comb, zcat, strep, zerosD, wout, zout,
    cidxa, cidxb, rowsa, rowsb, achunk, schunk, acc, semg, sems, semw,
):
    c = lax.axis_index("c")
    s = lax.axis_index("s")
    r0 = s * RPT
    pltpu.sync_copy(zerosD.at[pl.ds(r0, RPT)], acc.at[pl.ds(r0, RPT)])
    plsc.subcore_barrier()

    gbase = s * GPT

    def stage(cidx_c, rows_c, cidx_n, rows_n, m):
        @pl.when(m > 0)
        def _():
            for j in range(2):
                pltpu.make_async_copy(
                    rows_n.at[j], acc.at[cidx_n.at[j, 1]], sems
                ).wait()

        @pl.when(m < 2 * NPAIR - 1)
        def _():
            pltpu.sync_copy(comb.at[c, pl.ds(gbase + 2 * (m + 1), 2)], cidx_n)
            for j in range(2):
                pltpu.async_copy(zcat.at[cidx_n.at[j, 0]], rows_n.at[j], semg)

        for j in range(2):
            pltpu.make_async_copy(
                zcat.at[cidx_c.at[j, 0]], rows_c.at[j], semg
            ).wait()
        for j in range(2):
            pltpu.async_copy(rows_c.at[j], acc.at[cidx_c.at[j, 1]], sems, add=True)

    pltpu.sync_copy(comb.at[c, pl.ds(gbase, 2)], cidxa)
    for j in range(2):
        pltpu.async_copy(zcat.at[cidxa.at[j, 0]], rowsa.at[j], semg)

    def pair(k, carry):
        stage(cidxa, rowsa, cidxb, rowsb, 2 * k)
        stage(cidxb, rowsb, cidxa, rowsa, 2 * k + 1)
        return carry

    lax.fori_loop(0, NPAIR, pair, 0)
    for j in range(2):
        pltpu.make_async_copy(rowsb.at[j], acc.at[cidxb.at[j, 1]], sems).wait()
    plsc.subcore_barrier()

    wo = c * NP + r0
    hw = pltpu.async_copy(
        acc.at[pl.ds(r0, RPT)], wout.at[pl.ds(wo, RPT)], semw
    )

    def cchunk(q, carry):
        rq = r0 + q * CCH
        pltpu.sync_copy(acc.at[pl.ds(rq, CCH)], achunk)
        pltpu.sync_copy(strep.at[pl.ds(rq, CCH)], schunk)

        def rowf(r, carry2):
            sv = schunk[r, pl.ds(0, 16)]
            achunk[r, pl.ds(0, 16)] = achunk[r, pl.ds(0, 16)] * sv
            achunk[r, pl.ds(16, 16)] = achunk[r, pl.ds(16, 16)] * sv
            return carry2

        lax.fori_loop(0, CCH, rowf, 0)
        pltpu.sync_copy(achunk, zout.at[pl.ds(c * NP + rq, CCH)])
        return carry

    lax.fori_loop(0, RPT // CCH, cchunk, 0)
    hw.wait()


@functools.partial(
    pl.kernel,
    out_type=[
        jax.ShapeDtypeStruct((3 * B, D), _f32),
        jax.ShapeDtypeStruct((3 * B, D), _f32),
    ],
    mesh=_mesh,
    scratch_types=[
        pltpu.VMEM((3, 2, 128), jnp.int32),
        pltpu.VMEM((128, D), _f32),
        pltpu.VMEM((128, D), _f32),
        pltpu.VMEM((128, 16), _f32),
        pltpu.VMEM((3, 2, 128, DH), _f32),
        pltpu.SemaphoreType.DMA,
    ],
    compiler_params=_sc_params,
)
def _gather_kernel(
    embT, w1, w2, w3, trep, gidx3, rep_out, embr_out,
    idxb, ebuf, rbuf, tbuf, wbuf, sem,
):
    c = lax.axis_index("c")
    s = lax.axis_index("s")
    w = s * 2 + c
    pltpu.sync_copy(gidx3.at[pl.ds(w * 3, 3)], idxb)
    for j in range(3):
        o = (w * 3 + j) * 128
        hs = [pltpu.async_copy(embT.at[idxb.at[j, 0]], ebuf, sem)]
        hs.append(pltpu.async_copy(trep.at[idxb.at[j, 0]], tbuf, sem))
        for li, wt in enumerate((w1, w2, w3)):
            for h in range(2):
                hs.append(
                    pltpu.async_copy(wt.at[idxb.at[j, h]], wbuf.at[li, h], sem)
                )
        for h in hs:
            h.wait()

        def rowf(r, carry):
            tv = tbuf[r, pl.ds(0, 16)]
            for h in range(2):
                for k in range(2):
                    seg = pl.ds(h * DH + k * 16, 16)
                    sg = pl.ds(k * 16, 16)
                    wsv = (
                        wbuf[0, h, r, sg]
                        + wbuf[1, h, r, sg]
                        + wbuf[2, h, r, sg]
                    )
                    rbuf[r, seg] = (ebuf[r, seg] + tv * wsv) * 0.25
            return carry

        lax.fori_loop(0, 128, rowf, 0)
        pltpu.sync_copy(rbuf, rep_out.at[pl.ds(o, 128)])
        pltpu.sync_copy(ebuf, embr_out.at[pl.ds(o, 128)])


BM = 6272
NB = NP // BM


def _factor_body(ds_ref, dd_ref, emb_ref, zcat_ref, strep_ref, trep_ref):
    sv = lax.rsqrt(jnp.maximum(jnp.sum(ds_ref[0], axis=1, keepdims=True), 1.0))
    tv = lax.rsqrt(jnp.maximum(jnp.sum(dd_ref[0], axis=1, keepdims=True), 1.0))
    zcat_ref[...] = emb_ref[...] * sv
    stv = sv * tv
    strep_ref[...] = jnp.broadcast_to(stv, stv.shape[:1] + (16,))
    trep_ref[...] = jnp.broadcast_to(tv, tv.shape[:1] + (16,))


_factor_call = pl.pallas_call(
    _factor_body,
    grid=(2, NB),
    in_specs=[
        pl.BlockSpec((1, BM, 8), lambda h, i: (0, i, 0)),
        pl.BlockSpec((1, BM, 8), lambda h, i: (1, i, 0)),
        pl.BlockSpec((BM, DH), lambda h, i: (h * NB + i, 0)),
    ],
    out_specs=[
        pl.BlockSpec((BM, DH), lambda h, i: (h * NB + i, 0)),
        pl.BlockSpec((BM, 16), lambda h, i: (i, 0)),
        pl.BlockSpec((BM, 16), lambda h, i: (i, 0)),
    ],
    out_shape=[
        jax.ShapeDtypeStruct((2 * NP, DH), _f32),
        jax.ShapeDtypeStruct((NP, 16), _f32),
        jax.ShapeDtypeStruct((NP, 16), _f32),
    ],
)


def _l2_body(u_ref, p_ref, n_ref, out_ref):
    out_ref[...] = (
        jnp.sum(u_ref[...] * u_ref[...], axis=1, keepdims=True)
        + jnp.sum(p_ref[...] * p_ref[...], axis=1, keepdims=True)
        + jnp.sum(n_ref[...] * n_ref[...], axis=1, keepdims=True)
    )


_l2_call = pl.pallas_call(
    _l2_body,
    grid=(1,),
    in_specs=[
        pl.BlockSpec((B, D), lambda i: (0, 0)),
        pl.BlockSpec((B, D), lambda i: (1, 0)),
        pl.BlockSpec((B, D), lambda i: (2, 0)),
    ],
    out_specs=pl.BlockSpec((B, 1), lambda i: (0, 0)),
    out_shape=jax.ShapeDtypeStruct((B, 1), _f32),
)


def kernel(users, pos_items, neg_items, edge_index, embedding_weight):
    i32 = jnp.int32
    src = edge_index[0].astype(i32)
    dst = edge_index[1].astype(i32)
    pad = jnp.full((EP - E,), TRASH, dtype=i32)
    srcp = jnp.concatenate([src, pad])
    dstp = jnp.concatenate([dst, pad])
    edges2 = jnp.stack([srcp, dstp]).reshape(2, EP // 128, 128)
    srcg = srcp.reshape(EP // 128, 1, 128)
    dstg = dstp.reshape(EP // 128, 1, 128)
    comb = jnp.stack(
        [
            jnp.concatenate([srcg, dstg], axis=1),
            jnp.concatenate([srcg + NP, dstg], axis=1),
        ]
    )

    emb_p = jnp.concatenate(
        [embedding_weight.astype(_f32), jnp.zeros((NP - N, D), _f32)]
    )
    onesrow = jnp.zeros((128, 8), _f32).at[:, 0].set(1.0)
    zeros8 = jnp.zeros((NP, 8), _f32)
    zerosD = jnp.zeros((NP, DH), _f32)

    deg = _deg_kernel(edges2, onesrow, zeros8)

    emb_cat = jnp.concatenate([emb_p[:, :DH], emb_p[:, DH:]], axis=0)
    zcat, strep, trep = _factor_call(deg, deg, emb_cat)
    ws = []
    for _ in range(3):
        w, zcat = _spmm_kernel(comb, zcat, strep, zerosD)
        ws.append(w)

    gidx = jnp.concatenate(
        [
            users.astype(i32),
            NUSR + pos_items.astype(i32),
            NUSR + neg_items.astype(i32),
        ]
    ).reshape(3 * B // 128, 1, 128)
    gidx3 = jnp.concatenate([gidx, gidx + NP], axis=1)
    rep_rows, emb_rows = _gather_kernel(emb_p, ws[0], ws[1], ws[2], trep, gidx3)

    l2 = _l2_call(emb_rows, emb_rows, emb_rows).reshape(B)
    return (
        rep_rows[:B],
        rep_rows[B : 2 * B],
        rep_rows[2 * B :],
        l2,
    )
